# baseline (speedup 1.0000x reference)
import functools

import jax
import jax.numpy as jnp
from jax import lax
from jax.experimental import pallas as pl
from jax.experimental.pallas import tpu as pltpu
from jax.experimental.pallas import tpu_sc as plsc

from jax._src.pallas.mosaic import lowering as _mosaic_lowering
from jax._src.pallas.mosaic import core as _mosaic_core

_mosaic_lowering.lowering_rules[_mosaic_core.CoreType.SC_VECTOR_SUBCORE][lax.log_p] = (
    _mosaic_lowering.lowering_rules[_mosaic_core.CoreType.TC][lax.log_p])

B = 16384
C = 128
B_TC = 8192
B_SC = B - B_TC

_info = plsc.get_sparse_core_info()
_NC, _NS = _info.num_cores, _info.num_subcores
NW = _NC * _NS
CHUNK = 128
NCH_TC = B_TC // (NW * CHUNK)
NCH_SC = B_SC // (NW * CHUNK)


def _sc_gather(idx3, conf):
    mesh = plsc.VectorSubcoreMesh(core_axis_name="c", subcore_axis_name="s")

    @functools.partial(
        pl.kernel,
        mesh=mesh,
        out_type=jax.ShapeDtypeStruct((NW, NCH_TC, CHUNK, C), jnp.float32),
        scratch_types=[
            pltpu.VMEM((NCH_TC, CHUNK), jnp.int32),
            pltpu.VMEM((NCH_TC, CHUNK, C), jnp.float32),
            pltpu.SemaphoreType.DMA,
        ],
    )
    def k(idx_hbm, conf_hbm, out_hbm, idx_v, rows_v, sem):
        wid = lax.axis_index("s") * _NC + lax.axis_index("c")
        pltpu.sync_copy(idx_hbm.at[wid], idx_v)
        cps = [pltpu.async_copy(conf_hbm.at[idx_v.at[j]], rows_v.at[j], sem)
               for j in range(NCH_TC)]
        for cp in cps:
            cp.wait()
        pltpu.sync_copy(rows_v, out_hbm.at[wid])

    return k(idx3, conf)


def _tc_loss_sum(pred, target):
    BLK = 2048
    grid = B_TC // BLK

    def body(p_ref, t_ref, o_ref, acc_ref):
        @pl.when(pl.program_id(0) == 0)
        def _():
            acc_ref[0, 0] = 0.0

        acc_ref[0, 0] += jnp.sum(jnp.log(p_ref[...]) * t_ref[...])

        @pl.when(pl.program_id(0) == grid - 1)
        def _():
            o_ref[0, 0] = acc_ref[0, 0]

    return pl.pallas_call(
        body,
        grid=(grid,),
        in_specs=[
            pl.BlockSpec((BLK, C), lambda i: (i, 0)),
            pl.BlockSpec((BLK, C), lambda i: (i, 0)),
        ],
        out_specs=pl.BlockSpec(memory_space=pltpu.SMEM),
        out_shape=jax.ShapeDtypeStruct((1, 1), jnp.float32),
        scratch_shapes=[pltpu.SMEM((1, 1), jnp.float32)],
    )(pred, target)


def _sc_fused(idx3, pred4, conf):
    mesh = plsc.VectorSubcoreMesh(core_axis_name="c", subcore_axis_name="s")

    @functools.partial(
        pl.kernel,
        mesh=mesh,
        compiler_params=pltpu.CompilerParams(needs_layout_passes=False),
        out_type=jax.ShapeDtypeStruct((NW, 16), jnp.float32),
        scratch_types=[
            pltpu.VMEM((NCH_SC, CHUNK), jnp.int32),
            pltpu.VMEM((2, CHUNK, C), jnp.float32),
            pltpu.VMEM((2, CHUNK, C), jnp.float32),
            pltpu.VMEM((16,), jnp.float32),
            pltpu.SemaphoreType.DMA,
            pltpu.SemaphoreType.DMA,
            pltpu.SemaphoreType.DMA,
            pltpu.SemaphoreType.DMA,
        ],
    )
    def k(idx_hbm, pred_hbm, conf_hbm, out_hbm,
          idx_v, rows_v, pred_v, acc_v,
          gsem0, gsem1, psem0, psem1):
        wid = lax.axis_index("s") * _NC + lax.axis_index("c")
        pltpu.sync_copy(idx_hbm.at[wid], idx_v)
        gsems = (gsem0, gsem1)
        psems = (psem0, psem1)
        gcp = {0: pltpu.async_copy(conf_hbm.at[idx_v.at[0]], rows_v.at[0], gsem0)}
        pcp = {0: pltpu.async_copy(pred_hbm.at[wid, 0], pred_v.at[0], psem0)}
        accs = [jnp.zeros((16,), jnp.float32)] * (C // 16)
        for j in range(NCH_SC):
            if j + 1 < NCH_SC:
                nb = (j + 1) % 2
                gcp[j + 1] = pltpu.async_copy(
                    conf_hbm.at[idx_v.at[j + 1]], rows_v.at[nb], gsems[nb])
                pcp[j + 1] = pltpu.async_copy(
                    pred_hbm.at[wid, j + 1], pred_v.at[nb], psems[nb])
            gcp[j].wait()
            pcp[j].wait()
            buf = j % 2

            @plsc.parallel_loop(0, CHUNK, 2, carry=tuple(accs))
            def inner(r, accs, buf=buf):
                out = list(accs)
                for rr in range(2):
                    for c in range(C // 16):
                        t = rows_v[buf, r + rr, pl.ds(16 * c, 16)]
                        p = pred_v[buf, r + rr, pl.ds(16 * c, 16)]
                        out[c] = out[c] + t * jnp.log(p)
                return tuple(out)

            accs = inner
        acc = accs[0]
        for a in accs[1:]:
            acc = acc + a
        acc_v[...] = acc
        pltpu.sync_copy(acc_v, out_hbm.at[wid])

    return k(idx3, pred4, conf)


def kernel(classfy_out, index, confidence):
    idx_tc = index[:B_TC].reshape(NW, NCH_TC, CHUNK)
    idx_sc = index[B_TC:].reshape(NW, NCH_SC, CHUNK)
    pred_tc = classfy_out[:B_TC]
    pred_sc = classfy_out[B_TC:].reshape(NW, NCH_SC, CHUNK, C)
    target_tc = _sc_gather(idx_tc, confidence).reshape(B_TC, C)
    tc_sum = _tc_loss_sum(pred_tc, target_tc)
    sc_partials = _sc_fused(idx_sc, pred_sc, confidence)
    return -(tc_sum[0, 0] + jnp.sum(sc_partials)) / B


# speedup vs baseline: 1.2992x; 1.2992x over previous
import functools

import jax
import jax.numpy as jnp
from jax import lax
from jax.experimental import pallas as pl
from jax.experimental.pallas import tpu as pltpu
from jax.experimental.pallas import tpu_sc as plsc

from jax._src.pallas.mosaic import lowering as _mosaic_lowering
from jax._src.pallas.mosaic import core as _mosaic_core

_mosaic_lowering.lowering_rules[_mosaic_core.CoreType.SC_VECTOR_SUBCORE][lax.log_p] = (
    _mosaic_lowering.lowering_rules[_mosaic_core.CoreType.TC][lax.log_p])

B = 16384
C = 128

_info = plsc.get_sparse_core_info()
_NC, _NS = _info.num_cores, _info.num_subcores
NW = _NC * _NS
B_PER_W = B // NW
CHUNK = 128
N_CHUNK = B_PER_W // CHUNK


def _sc_fused(idx3, pred4, conf):
    mesh = plsc.VectorSubcoreMesh(core_axis_name="c", subcore_axis_name="s")

    @functools.partial(
        pl.kernel,
        mesh=mesh,
        compiler_params=pltpu.CompilerParams(needs_layout_passes=False),
        out_type=jax.ShapeDtypeStruct((NW, 16), jnp.float32),
        scratch_types=[
            pltpu.VMEM((N_CHUNK, CHUNK), jnp.int32),
            pltpu.VMEM((2, CHUNK, C), jnp.float32),
            pltpu.VMEM((2, CHUNK, C), jnp.float32),
            pltpu.VMEM((16,), jnp.float32),
            pltpu.SemaphoreType.DMA,
            pltpu.SemaphoreType.DMA,
            pltpu.SemaphoreType.DMA,
            pltpu.SemaphoreType.DMA,
        ],
    )
    def k(idx_hbm, pred_hbm, conf_hbm, out_hbm,
          idx_v, rows_v, pred_v, acc_v,
          gsem0, gsem1, psem0, psem1):
        wid = lax.axis_index("s") * _NC + lax.axis_index("c")
        pltpu.sync_copy(idx_hbm.at[wid], idx_v)
        gsems = (gsem0, gsem1)
        psems = (psem0, psem1)
        gcp = {0: pltpu.async_copy(conf_hbm.at[idx_v.at[0]], rows_v.at[0], gsem0)}
        pcp = {0: pltpu.async_copy(pred_hbm.at[wid, 0], pred_v.at[0], psem0)}
        accs = [jnp.zeros((16,), jnp.float32)] * (C // 16)
        for j in range(N_CHUNK):
            if j + 1 < N_CHUNK:
                nb = (j + 1) % 2
                gcp[j + 1] = pltpu.async_copy(
                    conf_hbm.at[idx_v.at[j + 1]], rows_v.at[nb], gsems[nb])
                pcp[j + 1] = pltpu.async_copy(
                    pred_hbm.at[wid, j + 1], pred_v.at[nb], psems[nb])
            gcp[j].wait()
            pcp[j].wait()
            buf = j % 2

            @plsc.parallel_loop(0, CHUNK, 2, carry=tuple(accs))
            def inner(r, accs, buf=buf):
                out = list(accs)
                for rr in range(2):
                    for c in range(C // 16):
                        t = rows_v[buf, r + rr, pl.ds(16 * c, 16)]
                        p = pred_v[buf, r + rr, pl.ds(16 * c, 16)]
                        out[c] = out[c] + t * jnp.log(p)
                return tuple(out)

            accs = inner
        acc = accs[0]
        for a in accs[1:]:
            acc = acc + a
        acc_v[...] = acc * (-1.0 / B)
        pltpu.sync_copy(acc_v, out_hbm.at[wid])

    return k(idx3, pred4, conf)


def kernel(classfy_out, index, confidence):
    idx3 = index.reshape(NW, N_CHUNK, CHUNK)
    pred4 = classfy_out.reshape(NW, N_CHUNK, CHUNK, C)
    partials = _sc_fused(idx3, pred4, confidence)
    return jnp.sum(partials)
